# baseline (speedup 1.0000x reference)
import functools

import jax
import jax.numpy as jnp
from jax import lax
from jax.experimental import pallas as pl
from jax.experimental.pallas import tpu as pltpu
from jax.experimental.pallas import tpu_sc as plsc

N = 10000
E = 320000
H = 128
STEPS = 8
B = 256

NW = 32
EPW = E // NW
CH = 128
NFULL = EPW // CH
CHT = EPW - NFULL * CH
RPT = 640
NPAD = 10240
RPW = NPAD // NW

@functools.cache
def _mesh():
    return plsc.VectorSubcoreMesh(core_axis_name="c", subcore_axis_name="s",
                                  num_cores=2, num_subcores=16)


def _sc_scatter_body(m_hbm, src_hbm, dst_hbm, out0, out1,
                     sic0, sic1, sic2, dic0, dic1, dic2, dtail_v,
                     rws0, rws1, rws2, agg_sh,
                     gs0, gs1, gs2, ss0, ss1, ss2, is0, is1, is2):
    sic = [sic0, sic1, sic2]
    dic = [dic0, dic1, dic2]
    rws = [rws0, rws1, rws2]
    gsem = [gs0, gs1, gs2]
    ssem = [ss0, ss1, ss2]
    isem = [is0, is1, is2]
    cid = lax.axis_index("c")
    sid = lax.axis_index("s")
    wid = cid * 16 + sid

    ebase = pl.multiple_of(wid * EPW, 8)

    def _idx_chunk(j, k):
        off = pl.multiple_of(ebase + j * CH, 8)
        pltpu.async_copy(src_hbm.at[pl.ds(off, CH)], sic[k], isem[k])
        pltpu.async_copy(dst_hbm.at[pl.ds(off, CH)], dic[k], isem[k])

    def _idx_wait(k):
        pltpu.make_async_copy(src_hbm.at[pl.ds(0, CH)], sic[k],
                              isem[k]).wait()
        pltpu.make_async_copy(dst_hbm.at[pl.ds(0, CH)], dic[k],
                              isem[k]).wait()

    def _gather(k):
        return pltpu.make_async_copy(m_hbm.at[sic[k]], rws[k], gsem[k])

    def _scatter(k):
        return pltpu.make_async_copy(rws[k], agg_sh.at[dic[k]], ssem[k])

    _idx_chunk(0, 0)
    _idx_chunk(1, 1)

    zf = jnp.zeros((16,), jnp.float32)

    def _z(i, _):
        for v in range(8):
            rws2[i, pl.ds(16 * v, 16)] = zf
        return 0

    lax.fori_loop(0, CH, _z, 0)
    _idx_wait(0)
    _gather(0).start()
    zbase = pl.multiple_of(sid * RPT, 8)

    @pl.when(sid < 15)
    def _():
        for k in range(RPT // CH):
            pltpu.sync_copy(rws2, agg_sh.at[pl.ds(zbase + k * CH, CH)])

    @pl.when(sid == 15)
    def _():
        for k in range(3):
            pltpu.sync_copy(rws2, agg_sh.at[pl.ds(zbase + k * CH, CH)])
        pltpu.sync_copy(rws2.at[pl.ds(0, CHT)],
                        agg_sh.at[pl.ds(zbase + 3 * CH, CHT)])

    plsc.subcore_barrier()

    def _substep(j, k, jj=None):
        kn = (k + 1) % 3
        kp = (k + 2) % 3
        _idx_wait(kn)
        _gather(kn).start()
        if jj is None:
            _scatter(kp).wait()
        else:
            @pl.when(jj >= 1)
            def _():
                _scatter(kp).wait()

        _idx_chunk(j + 2, kp)
        _gather(k).wait()
        pltpu.async_copy(rws[k], agg_sh.at[dic[k]], ssem[k], add=True)

    def _pipe(jj, _):
        j = 3 * jj
        _substep(j, 0, jj=jj)
        _substep(j + 1, 1)
        _substep(j + 2, 2)
        return 0

    lax.fori_loop(0, 25, _pipe, 0)

    _substep(75, 0)

    _idx_wait(2)
    _gather(2).start()
    _scatter(0).wait()
    toff = pl.multiple_of(ebase + NFULL * CH, 8)
    pltpu.async_copy(src_hbm.at[pl.ds(toff, CHT)],
                     sic[0].at[pl.ds(0, CHT)], isem[0])
    pltpu.async_copy(dst_hbm.at[pl.ds(toff, CHT)], dtail_v, isem[0])
    _gather(1).wait()
    pltpu.async_copy(rws[1], agg_sh.at[dic[1]], ssem[1], add=True)

    pltpu.make_async_copy(src_hbm.at[pl.ds(0, CHT)],
                          sic[0].at[pl.ds(0, CHT)], isem[0]).wait()
    pltpu.make_async_copy(dst_hbm.at[pl.ds(0, CHT)], dtail_v, isem[0]).wait()
    pltpu.async_copy(m_hbm.at[sic[0].at[pl.ds(0, CHT)]],
                     rws[0].at[pl.ds(0, CHT)], gsem[0])
    _scatter(1).wait()
    _gather(2).wait()
    pltpu.async_copy(rws[2], agg_sh.at[dic[2]], ssem[2], add=True)

    pltpu.make_async_copy(m_hbm.at[sic[0].at[pl.ds(0, CHT)]],
                          rws[0].at[pl.ds(0, CHT)], gsem[0]).wait()
    _scatter(2).wait()
    pltpu.sync_copy(rws[0].at[pl.ds(0, CHT)], agg_sh.at[dtail_v], add=True)
    plsc.subcore_barrier()

    obase = pl.multiple_of(sid * RPT, 8)
    nout = N - 15 * RPT

    @pl.when(jnp.logical_and(cid == 0, sid < 15))
    def _():
        pltpu.sync_copy(agg_sh.at[pl.ds(obase, RPT)], out0.at[pl.ds(obase, RPT)])

    @pl.when(jnp.logical_and(cid == 0, sid == 15))
    def _():
        pltpu.sync_copy(agg_sh.at[pl.ds(obase, nout)], out0.at[pl.ds(obase, nout)])

    @pl.when(jnp.logical_and(cid == 1, sid < 15))
    def _():
        pltpu.sync_copy(agg_sh.at[pl.ds(obase, RPT)], out1.at[pl.ds(obase, RPT)])

    @pl.when(jnp.logical_and(cid == 1, sid == 15))
    def _():
        pltpu.sync_copy(agg_sh.at[pl.ds(obase, nout)], out1.at[pl.ds(obase, nout)])


@functools.cache
def _sc_scatter():
    return pl.kernel(
        _sc_scatter_body,
        out_type=(jax.ShapeDtypeStruct((N, H), jnp.float32),
                  jax.ShapeDtypeStruct((N, H), jnp.float32)),
        mesh=_mesh(),
        scratch_types=(
            [pltpu.VMEM((CH,), jnp.int32)] * 6
            + [pltpu.VMEM((CHT,), jnp.int32)]
            + [pltpu.VMEM((CH, H), jnp.float32)] * 3
            + [pltpu.VMEM_SHARED((N, H), jnp.float32)]
            + [pltpu.SemaphoreType.DMA] * 9
        ),
    )


def _sc_pool_body(hx_hbm, bat_hbm, out_hbm, rows_v, bat_v, acc_v):
    cid = lax.axis_index("c")
    sid = lax.axis_index("s")
    wid = cid * 16 + sid
    base = pl.multiple_of(wid * RPW, 8)
    ntail = N - 31 * RPW

    @pl.when(wid < 31)
    def _():
        pltpu.sync_copy(hx_hbm.at[pl.ds(base, RPW)], rows_v)
        pltpu.sync_copy(bat_hbm.at[pl.ds(base, RPW)], bat_v.at[pl.ds(0, RPW)])

    @pl.when(wid == 31)
    def _():
        pltpu.sync_copy(hx_hbm.at[pl.ds(base, ntail)],
                        rows_v.at[pl.ds(0, ntail)])
        pltpu.sync_copy(bat_hbm.at[pl.ds(base, ntail)],
                        bat_v.at[pl.ds(0, ntail)])

    ninf = jnp.full((16,), -jnp.inf, jnp.float32)

    def _init(i, _):
        for v in range(8):
            acc_v[i, pl.ds(16 * v, 16)] = ninf
        return 0

    lax.fori_loop(0, B, _init, 0)

    def _scan(r, _):
        b = bat_v[pl.ds(r, 16)][0]
        for v in range(8):
            cur = acc_v[b, pl.ds(16 * v, 16)]
            row = rows_v[r, pl.ds(16 * v, 16)]
            acc_v[b, pl.ds(16 * v, 16)] = jnp.maximum(cur, row)
        return 0

    nscan = jnp.where(wid == 31, ntail, RPW)
    lax.fori_loop(0, nscan, _scan, 0)
    pltpu.sync_copy(acc_v, out_hbm.at[wid])


@functools.cache
def _sc_pool():
    return pl.kernel(
        _sc_pool_body,
        out_type=jax.ShapeDtypeStruct((NW, B, H), jnp.float32),
        mesh=_mesh(),
        scratch_types=[
            pltpu.VMEM((RPW, H), jnp.float32),
            pltpu.VMEM((RPW + 16,), jnp.int32),
            pltpu.VMEM((B, H), jnp.float32),
        ],
    )


_BLK = 10000
_GRID = N // _BLK


def _mm_body(x_ref, w_ref, o_ref):
    o_ref[...] = jnp.dot(x_ref[...], w_ref[...],
                         preferred_element_type=jnp.float32)


def _mm(x, w):
    return pl.pallas_call(
        _mm_body,
        grid=(_GRID,),
        in_specs=[pl.BlockSpec((_BLK, H), lambda i: (i, 0)),
                  pl.BlockSpec((H, H), lambda i: (0, 0))],
        out_specs=pl.BlockSpec((_BLK, H), lambda i: (i, 0)),
        out_shape=jax.ShapeDtypeStruct((N, H), jnp.float32),
    )(x, w)


def _gru_math(h, agg, wih_t, whh_t, bih, bhh):
    gi = jnp.dot(agg, wih_t, preferred_element_type=jnp.float32) + bih
    gh = jnp.dot(h, whh_t, preferred_element_type=jnp.float32) + bhh
    r = jax.nn.sigmoid(gi[:, :H] + gh[:, :H])
    z = jax.nn.sigmoid(gi[:, H:2 * H] + gh[:, H:2 * H])
    n = jnp.tanh(gi[:, 2 * H:] + r * gh[:, 2 * H:])
    return (1.0 - z) * n + z * h


def _gru_step_body(h_ref, p0_ref, p1_ref, wih_ref, whh_ref, bih_ref, bhh_ref,
                   wnx_ref, h_out, m_out):
    hn = _gru_math(h_ref[...], p0_ref[...] + p1_ref[...], wih_ref[...],
                   whh_ref[...], bih_ref[...], bhh_ref[...])
    h_out[...] = hn
    m_out[...] = jnp.dot(hn, wnx_ref[...], preferred_element_type=jnp.float32)


def _gru_step(h, p0, p1, wih_t, whh_t, bih, bhh, wnx):
    blk = lambda i: (i, 0)
    full = lambda i: (0, 0)
    return pl.pallas_call(
        _gru_step_body,
        grid=(_GRID,),
        in_specs=[pl.BlockSpec((_BLK, H), blk),
                  pl.BlockSpec((_BLK, H), blk),
                  pl.BlockSpec((_BLK, H), blk),
                  pl.BlockSpec((H, 3 * H), full),
                  pl.BlockSpec((H, 3 * H), full),
                  pl.BlockSpec((1, 3 * H), full),
                  pl.BlockSpec((1, 3 * H), full),
                  pl.BlockSpec((H, H), full)],
        out_specs=(pl.BlockSpec((_BLK, H), blk), pl.BlockSpec((_BLK, H), blk)),
        out_shape=(jax.ShapeDtypeStruct((N, H), jnp.float32),
                   jax.ShapeDtypeStruct((N, H), jnp.float32)),
    )(h, p0, p1, wih_t, whh_t, bih, bhh, wnx)


def _gru_final_body(h_ref, p0_ref, p1_ref, wih_ref, whh_ref, bih_ref,
                    bhh_ref, hx_out):
    hn = _gru_math(h_ref[...], p0_ref[...] + p1_ref[...], wih_ref[...],
                   whh_ref[...], bih_ref[...], bhh_ref[...])
    hx_out[...] = jnp.maximum(hn, 0.0)


def _gru_final(h, p0, p1, wih_t, whh_t, bih, bhh):
    blk = lambda i: (i, 0)
    full = lambda i: (0, 0)
    return pl.pallas_call(
        _gru_final_body,
        grid=(_GRID,),
        in_specs=[pl.BlockSpec((_BLK, H), blk),
                  pl.BlockSpec((_BLK, H), blk),
                  pl.BlockSpec((_BLK, H), blk),
                  pl.BlockSpec((H, 3 * H), full),
                  pl.BlockSpec((H, 3 * H), full),
                  pl.BlockSpec((1, 3 * H), full),
                  pl.BlockSpec((1, 3 * H), full)],
        out_specs=pl.BlockSpec((_BLK, H), blk),
        out_shape=jax.ShapeDtypeStruct((N, H), jnp.float32),
    )(h, p0, p1, wih_t, whh_t, bih, bhh)


def _head_body(parts_ref, c1_ref, b1_ref, c2_ref, b2_ref, f1_ref, fb1_ref,
               f2_ref, fb2_ref, o_ref):
    pooled = jnp.max(parts_ref[...], axis=0)
    t = jnp.maximum(jnp.dot(pooled, c1_ref[...],
                            preferred_element_type=jnp.float32) + b1_ref[...],
                    0.0)
    t = jnp.maximum(jnp.dot(t, c2_ref[...],
                            preferred_element_type=jnp.float32) + b2_ref[...],
                    0.0)
    t = jnp.maximum(jnp.dot(t, f1_ref[...],
                            preferred_element_type=jnp.float32) + fb1_ref[...],
                    0.0)
    o_ref[...] = jnp.dot(t, f2_ref[...],
                         preferred_element_type=jnp.float32) + fb2_ref[...]


def _head(parts, c1t, b1, c2t, b2, f1t, fb1, f2t, fb2):
    return pl.pallas_call(
        _head_body,
        out_shape=jax.ShapeDtypeStruct((B, 2), jnp.float32),
    )(parts, c1t, b1, c2t, b2, f1t, fb1, f2t, fb2)


def kernel(x, edge_index, batch, ggc_w, gru_w_ih, gru_w_hh, gru_b_ih,
           gru_b_hh, conv1_w, conv1_b, conv2_w, conv2_b, fc1_w, fc1_b,
           fc2_w, fc2_b):
    src = edge_index[0]
    dst = edge_index[1]
    wih_t = gru_w_ih.T
    whh_t = gru_w_hh.T
    bih = gru_b_ih.reshape(1, 3 * H)
    bhh = gru_b_hh.reshape(1, 3 * H)

    h = x
    m = _mm(x, ggc_w[0])
    for i in range(STEPS):
        p0, p1 = _sc_scatter()(m, src, dst)
        if i + 1 < STEPS:
            h, m = _gru_step(h, p0, p1, wih_t, whh_t, bih, bhh, ggc_w[i + 1])
        else:
            hx = _gru_final(h, p0, p1, wih_t, whh_t, bih, bhh)

    parts = _sc_pool()(hx, batch)

    out = _head(parts,
                conv1_w[:, :, 1].T, conv1_b.reshape(1, H),
                conv2_w[:, :, 1].T, conv2_b.reshape(1, H),
                fc1_w.T, fc1_b.reshape(1, H // 2),
                fc2_w.T, fc2_b.reshape(1, 2))
    return out


# speedup vs baseline: 1.0218x; 1.0218x over previous
import functools

import jax
import jax.numpy as jnp
from jax import lax
from jax.experimental import pallas as pl
from jax.experimental.pallas import tpu as pltpu
from jax.experimental.pallas import tpu_sc as plsc

N = 10000
E = 320000
H = 128
STEPS = 8
B = 256

NW = 32
EPW = E // NW
CH = 128
NFULL = EPW // CH
CHT = EPW - NFULL * CH
RPT = 640
NPAD = 10240
RPW = NPAD // NW

@functools.cache
def _mesh():
    return plsc.VectorSubcoreMesh(core_axis_name="c", subcore_axis_name="s",
                                  num_cores=2, num_subcores=16)


def _sc_scatter_body(m_hbm, src_hbm, dst_hbm, out0, out1,
                     sic0, sic1, sic2, dic0, dic1, dic2, dtail_v,
                     rws0, rws1, rws2, agg_sh,
                     gs0, gs1, gs2, ss0, ss1, ss2, is0, is1, is2):
    sic = [sic0, sic1, sic2]
    dic = [dic0, dic1, dic2]
    rws = [rws0, rws1, rws2]
    gsem = [gs0, gs1, gs2]
    ssem = [ss0, ss1, ss2]
    isem = [is0, is1, is2]
    cid = lax.axis_index("c")
    sid = lax.axis_index("s")
    wid = cid * 16 + sid

    ebase = pl.multiple_of(wid * EPW, 8)

    def _idx_chunk(j, k):
        off = pl.multiple_of(ebase + j * CH, 8)
        pltpu.async_copy(src_hbm.at[pl.ds(off, CH)], sic[k], isem[k])
        pltpu.async_copy(dst_hbm.at[pl.ds(off, CH)], dic[k], isem[k])

    def _idx_wait(k):
        pltpu.make_async_copy(src_hbm.at[pl.ds(0, CH)], sic[k],
                              isem[k]).wait()
        pltpu.make_async_copy(dst_hbm.at[pl.ds(0, CH)], dic[k],
                              isem[k]).wait()

    def _gather(k):
        return pltpu.make_async_copy(m_hbm.at[sic[k]], rws[k], gsem[k])

    def _scatter(k):
        return pltpu.make_async_copy(rws[k], agg_sh.at[dic[k]], ssem[k])

    _idx_chunk(0, 0)
    _idx_chunk(1, 1)

    zf = jnp.zeros((16,), jnp.float32)

    def _z(i, _):
        for v in range(8):
            rws2[i, pl.ds(16 * v, 16)] = zf
        return 0

    lax.fori_loop(0, CH, _z, 0)
    _idx_wait(0)
    _gather(0).start()
    zbase = pl.multiple_of(sid * RPT, 8)

    @pl.when(sid < 15)
    def _():
        for k in range(RPT // CH):
            pltpu.sync_copy(rws2, agg_sh.at[pl.ds(zbase + k * CH, CH)])

    @pl.when(sid == 15)
    def _():
        for k in range(3):
            pltpu.sync_copy(rws2, agg_sh.at[pl.ds(zbase + k * CH, CH)])
        pltpu.sync_copy(rws2.at[pl.ds(0, CHT)],
                        agg_sh.at[pl.ds(zbase + 3 * CH, CHT)])

    plsc.subcore_barrier()

    def _substep(j, k, jj=None):
        kn = (k + 1) % 3
        kp = (k + 2) % 3
        _idx_wait(kn)
        _gather(kn).start()
        if jj is None:
            _scatter(kp).wait()
        else:
            @pl.when(jj >= 1)
            def _():
                _scatter(kp).wait()

        _idx_chunk(j + 2, kp)
        _gather(k).wait()
        pltpu.async_copy(rws[k], agg_sh.at[dic[k]], ssem[k], add=True)

    def _pipe(jj, _):
        j = 3 * jj
        _substep(j, 0, jj=jj)
        _substep(j + 1, 1)
        _substep(j + 2, 2)
        return 0

    lax.fori_loop(0, 25, _pipe, 0)

    _substep(75, 0)

    _idx_wait(2)
    _gather(2).start()
    _scatter(0).wait()
    toff = pl.multiple_of(ebase + NFULL * CH, 8)
    pltpu.async_copy(src_hbm.at[pl.ds(toff, CHT)],
                     sic[0].at[pl.ds(0, CHT)], isem[0])
    pltpu.async_copy(dst_hbm.at[pl.ds(toff, CHT)], dtail_v, isem[0])
    _gather(1).wait()
    pltpu.async_copy(rws[1], agg_sh.at[dic[1]], ssem[1], add=True)

    pltpu.make_async_copy(src_hbm.at[pl.ds(0, CHT)],
                          sic[0].at[pl.ds(0, CHT)], isem[0]).wait()
    pltpu.make_async_copy(dst_hbm.at[pl.ds(0, CHT)], dtail_v, isem[0]).wait()
    pltpu.async_copy(m_hbm.at[sic[0].at[pl.ds(0, CHT)]],
                     rws[0].at[pl.ds(0, CHT)], gsem[0])
    _scatter(1).wait()
    _gather(2).wait()
    pltpu.async_copy(rws[2], agg_sh.at[dic[2]], ssem[2], add=True)

    pltpu.make_async_copy(m_hbm.at[sic[0].at[pl.ds(0, CHT)]],
                          rws[0].at[pl.ds(0, CHT)], gsem[0]).wait()
    _scatter(2).wait()
    pltpu.sync_copy(rws[0].at[pl.ds(0, CHT)], agg_sh.at[dtail_v], add=True)
    plsc.subcore_barrier()

    obase = pl.multiple_of(sid * RPT, 8)
    nout = N - 15 * RPT

    @pl.when(jnp.logical_and(cid == 0, sid < 15))
    def _():
        pltpu.sync_copy(agg_sh.at[pl.ds(obase, RPT)], out0.at[pl.ds(obase, RPT)])

    @pl.when(jnp.logical_and(cid == 0, sid == 15))
    def _():
        pltpu.sync_copy(agg_sh.at[pl.ds(obase, nout)], out0.at[pl.ds(obase, nout)])

    @pl.when(jnp.logical_and(cid == 1, sid < 15))
    def _():
        pltpu.sync_copy(agg_sh.at[pl.ds(obase, RPT)], out1.at[pl.ds(obase, RPT)])

    @pl.when(jnp.logical_and(cid == 1, sid == 15))
    def _():
        pltpu.sync_copy(agg_sh.at[pl.ds(obase, nout)], out1.at[pl.ds(obase, nout)])


@functools.cache
def _sc_scatter():
    return pl.kernel(
        _sc_scatter_body,
        out_type=(jax.ShapeDtypeStruct((N, H), jnp.float32),
                  jax.ShapeDtypeStruct((N, H), jnp.float32)),
        mesh=_mesh(),
        scratch_types=(
            [pltpu.VMEM((CH,), jnp.int32)] * 6
            + [pltpu.VMEM((CHT,), jnp.int32)]
            + [pltpu.VMEM((CH, H), jnp.float32)] * 3
            + [pltpu.VMEM_SHARED((N, H), jnp.float32)]
            + [pltpu.SemaphoreType.DMA] * 9
        ),
    )


def _sc_pool_body(hx_hbm, bat_hbm, out_hbm, rows_v, bat_v, acc_v):
    cid = lax.axis_index("c")
    sid = lax.axis_index("s")
    wid = cid * 16 + sid
    base = pl.multiple_of(wid * RPW, 8)
    ntail = N - 31 * RPW

    @pl.when(wid < 31)
    def _():
        pltpu.sync_copy(hx_hbm.at[pl.ds(base, RPW)], rows_v)
        pltpu.sync_copy(bat_hbm.at[pl.ds(base, RPW)], bat_v.at[pl.ds(0, RPW)])

    @pl.when(wid == 31)
    def _():
        pltpu.sync_copy(hx_hbm.at[pl.ds(base, ntail)],
                        rows_v.at[pl.ds(0, ntail)])
        pltpu.sync_copy(bat_hbm.at[pl.ds(base, ntail)],
                        bat_v.at[pl.ds(0, ntail)])

    ninf = jnp.full((16,), -jnp.inf, jnp.float32)

    def _init(i, _):
        for v in range(8):
            acc_v[i, pl.ds(16 * v, 16)] = ninf
        return 0

    lax.fori_loop(0, B, _init, 0)

    def _scan(r, _):
        b = bat_v[pl.ds(r, 16)][0]
        for v in range(8):
            cur = acc_v[b, pl.ds(16 * v, 16)]
            row = rows_v[r, pl.ds(16 * v, 16)]
            acc_v[b, pl.ds(16 * v, 16)] = jnp.maximum(cur, row)
        return 0

    nscan = jnp.where(wid == 31, ntail, RPW)
    lax.fori_loop(0, nscan, _scan, 0)
    pltpu.sync_copy(acc_v, out_hbm.at[wid])


@functools.cache
def _sc_pool():
    return pl.kernel(
        _sc_pool_body,
        out_type=jax.ShapeDtypeStruct((NW, B, H), jnp.float32),
        mesh=_mesh(),
        scratch_types=[
            pltpu.VMEM((RPW, H), jnp.float32),
            pltpu.VMEM((RPW + 16,), jnp.int32),
            pltpu.VMEM((B, H), jnp.float32),
        ],
    )


_BLK = 2000
_GRID = N // _BLK


def _mm_body(x_ref, w_ref, o_ref):
    o_ref[...] = jnp.dot(x_ref[...], w_ref[...],
                         preferred_element_type=jnp.float32)


def _mm(x, w):
    return pl.pallas_call(
        _mm_body,
        grid=(_GRID,),
        in_specs=[pl.BlockSpec((_BLK, H), lambda i: (i, 0)),
                  pl.BlockSpec((H, H), lambda i: (0, 0))],
        out_specs=pl.BlockSpec((_BLK, H), lambda i: (i, 0)),
        out_shape=jax.ShapeDtypeStruct((N, H), jnp.float32),
    )(x, w)


def _gru_math(h, agg, wih_t, whh_t, bih, bhh):
    gi = jnp.dot(agg, wih_t, preferred_element_type=jnp.float32) + bih
    gh = jnp.dot(h, whh_t, preferred_element_type=jnp.float32) + bhh
    r = jax.nn.sigmoid(gi[:, :H] + gh[:, :H])
    z = jax.nn.sigmoid(gi[:, H:2 * H] + gh[:, H:2 * H])
    n = jnp.tanh(gi[:, 2 * H:] + r * gh[:, 2 * H:])
    return (1.0 - z) * n + z * h


def _gru_step_body(h_ref, p0_ref, p1_ref, wih_ref, whh_ref, bih_ref, bhh_ref,
                   wnx_ref, h_out, m_out):
    hn = _gru_math(h_ref[...], p0_ref[...] + p1_ref[...], wih_ref[...],
                   whh_ref[...], bih_ref[...], bhh_ref[...])
    h_out[...] = hn
    m_out[...] = jnp.dot(hn, wnx_ref[...], preferred_element_type=jnp.float32)


def _gru_step(h, p0, p1, wih_t, whh_t, bih, bhh, wnx):
    blk = lambda i: (i, 0)
    full = lambda i: (0, 0)
    return pl.pallas_call(
        _gru_step_body,
        grid=(_GRID,),
        in_specs=[pl.BlockSpec((_BLK, H), blk),
                  pl.BlockSpec((_BLK, H), blk),
                  pl.BlockSpec((_BLK, H), blk),
                  pl.BlockSpec((H, 3 * H), full),
                  pl.BlockSpec((H, 3 * H), full),
                  pl.BlockSpec((1, 3 * H), full),
                  pl.BlockSpec((1, 3 * H), full),
                  pl.BlockSpec((H, H), full)],
        out_specs=(pl.BlockSpec((_BLK, H), blk), pl.BlockSpec((_BLK, H), blk)),
        out_shape=(jax.ShapeDtypeStruct((N, H), jnp.float32),
                   jax.ShapeDtypeStruct((N, H), jnp.float32)),
    )(h, p0, p1, wih_t, whh_t, bih, bhh, wnx)


def _gru_final_body(h_ref, p0_ref, p1_ref, wih_ref, whh_ref, bih_ref,
                    bhh_ref, hx_out):
    hn = _gru_math(h_ref[...], p0_ref[...] + p1_ref[...], wih_ref[...],
                   whh_ref[...], bih_ref[...], bhh_ref[...])
    hx_out[...] = jnp.maximum(hn, 0.0)


def _gru_final(h, p0, p1, wih_t, whh_t, bih, bhh):
    blk = lambda i: (i, 0)
    full = lambda i: (0, 0)
    return pl.pallas_call(
        _gru_final_body,
        grid=(_GRID,),
        in_specs=[pl.BlockSpec((_BLK, H), blk),
                  pl.BlockSpec((_BLK, H), blk),
                  pl.BlockSpec((_BLK, H), blk),
                  pl.BlockSpec((H, 3 * H), full),
                  pl.BlockSpec((H, 3 * H), full),
                  pl.BlockSpec((1, 3 * H), full),
                  pl.BlockSpec((1, 3 * H), full)],
        out_specs=pl.BlockSpec((_BLK, H), blk),
        out_shape=jax.ShapeDtypeStruct((N, H), jnp.float32),
    )(h, p0, p1, wih_t, whh_t, bih, bhh)


def _head_body(parts_ref, c1_ref, b1_ref, c2_ref, b2_ref, f1_ref, fb1_ref,
               f2_ref, fb2_ref, o_ref):
    pooled = jnp.max(parts_ref[...], axis=0)
    t = jnp.maximum(jnp.dot(pooled, c1_ref[...],
                            preferred_element_type=jnp.float32) + b1_ref[...],
                    0.0)
    t = jnp.maximum(jnp.dot(t, c2_ref[...],
                            preferred_element_type=jnp.float32) + b2_ref[...],
                    0.0)
    t = jnp.maximum(jnp.dot(t, f1_ref[...],
                            preferred_element_type=jnp.float32) + fb1_ref[...],
                    0.0)
    o_ref[...] = jnp.dot(t, f2_ref[...],
                         preferred_element_type=jnp.float32) + fb2_ref[...]


def _head(parts, c1t, b1, c2t, b2, f1t, fb1, f2t, fb2):
    return pl.pallas_call(
        _head_body,
        out_shape=jax.ShapeDtypeStruct((B, 2), jnp.float32),
    )(parts, c1t, b1, c2t, b2, f1t, fb1, f2t, fb2)


def kernel(x, edge_index, batch, ggc_w, gru_w_ih, gru_w_hh, gru_b_ih,
           gru_b_hh, conv1_w, conv1_b, conv2_w, conv2_b, fc1_w, fc1_b,
           fc2_w, fc2_b):
    src = edge_index[0]
    dst = edge_index[1]
    wih_t = gru_w_ih.T
    whh_t = gru_w_hh.T
    bih = gru_b_ih.reshape(1, 3 * H)
    bhh = gru_b_hh.reshape(1, 3 * H)

    h = x
    m = _mm(x, ggc_w[0])
    for i in range(STEPS):
        p0, p1 = _sc_scatter()(m, src, dst)
        if i + 1 < STEPS:
            h, m = _gru_step(h, p0, p1, wih_t, whh_t, bih, bhh, ggc_w[i + 1])
        else:
            hx = _gru_final(h, p0, p1, wih_t, whh_t, bih, bhh)

    parts = _sc_pool()(hx, batch)

    out = _head(parts,
                conv1_w[:, :, 1].T, conv1_b.reshape(1, H),
                conv2_w[:, :, 1].T, conv2_b.reshape(1, H),
                fc1_w.T, fc1_b.reshape(1, H // 2),
                fc2_w.T, fc2_b.reshape(1, 2))
    return out
